# baseline (speedup 1.0000x reference)
import functools
import math

import jax
import jax.numpy as jnp
from jax import lax
from jax.experimental import pallas as pl
from jax.experimental.pallas import tpu as pltpu

_N_STEPS = 16
_PATCH_SIZE = 4
_PATCH_STRIDE = 4
_PATCH_NUMS = int((_N_STEPS - _PATCH_SIZE) / _PATCH_STRIDE + 2)
_PROMPT_LEN = 8
_D_LLM = 512
_D_FFN = 32
_N_HEADS = 4
_LLM_HEADS = 4
_N_LAYERS = 2

_BF = jnp.bfloat16
_F32 = jnp.float32


def _ln(x, g, b, eps=1e-5):
    mu = jnp.mean(x, axis=-1, keepdims=True)
    var = jnp.mean((x - mu) ** 2, axis=-1, keepdims=True)
    return (x - mu) * lax.rsqrt(var + eps) * g + b


def _softmax_rows(s):
    m = jnp.max(s, axis=-1, keepdims=True)
    p = jnp.exp(s - m)
    return p * pl.reciprocal(jnp.sum(p, axis=-1, keepdims=True), approx=True)


def _full_spec(arr):
    return pl.BlockSpec(arr.shape, lambda *ij, _n=arr.ndim: (0,) * _n)


def _kv_kernel(map_w_ref, word_emb_ref, map_b_ref, rkv_w_ref, rkv_b_ref,
               kv_ref):
    src = jnp.dot(map_w_ref[...], word_emb_ref[...],
                  preferred_element_type=_F32) + map_b_ref[...]
    kv = jnp.dot(src.astype(_BF), rkv_w_ref[...],
                 preferred_element_type=_F32) + rkv_b_ref[...]
    kv_ref[...] = kv.astype(_BF)


def _kv_call(map_w, word_emb, map_b, rkv_w, rkv_b):
    num_tok = map_w.shape[0]
    dqk2 = rkv_w.shape[1]
    ins = [map_w.astype(_BF), word_emb.astype(_BF), map_b,
           rkv_w.astype(_BF), rkv_b]
    return pl.pallas_call(
        _kv_kernel,
        out_shape=jax.ShapeDtypeStruct((num_tok, dqk2), _BF),
        grid=(1,),
        in_specs=[_full_spec(a) for a in ins],
        out_specs=pl.BlockSpec((num_tok, dqk2), lambda i: (0, 0)),
        compiler_params=pltpu.CompilerParams(dimension_semantics=("arbitrary",)),
    )(*ins)


def _fused_kernel(xt_ref, mk_ref, vemb_ref, rq_w_ref, rq_b_ref, kv_ref,
                  ro_w_ref, ro_b_ref, pwp_ref, te_ref,
                  attn_w_ref, attn_b_ref, proj_w_ref, proj_b_ref,
                  ln1_g_ref, ln1_b_ref, ln2_g_ref, ln2_b_ref,
                  fc_w_ref, fc_b_ref, mlp_w_ref, mlp_b_ref,
                  lnf_g_ref, lnf_b_ref, head_w_ref, head_b_ref, sel_ref,
                  out_ref, *, n_heads, d_keys, n_layers, llm_heads, d_llm,
                  d_ffn, patch_nums, G, S, T):
    X = xt_ref[...]
    mk = mk_ref[...]

    cnt = jnp.maximum(jnp.sum(mk, axis=-1, keepdims=True), 1.0)
    mean = jnp.sum(X * mk, axis=-1, keepdims=True) / cnt
    xn = (X - mean) * mk
    std = jnp.sqrt(jnp.sum(xn * xn, axis=-1, keepdims=True) / cnt + 1e-5)
    xn = xn / std

    pats = [xn[:, p * _PATCH_STRIDE: p * _PATCH_STRIDE + _PATCH_SIZE]
            for p in range(patch_nums - 1)]
    last = jnp.broadcast_to(xn[:, T - 1:T], (G, _PATCH_SIZE))
    pats.append(last)
    patch = jnp.concatenate(pats, axis=0)

    vemb = vemb_ref[...]
    enc = patch[:, 0:1] * vemb[0:1, :]
    for j in range(1, _PATCH_SIZE):
        enc = enc + patch[:, j:j + 1] * vemb[j:j + 1, :]
    q = jnp.dot(enc.astype(_BF), rq_w_ref[...],
                preferred_element_type=_F32) + rq_b_ref[...]

    kv = kv_ref[...]
    hdk = n_heads * d_keys
    scale = 1.0 / math.sqrt(d_keys)
    ro_w = ro_w_ref[...]
    rep = jnp.zeros((patch_nums * G, d_llm), _F32)
    for h in range(n_heads):
        sl = slice(h * d_keys, (h + 1) * d_keys)
        qh = (q[:, sl] * scale).astype(_BF)
        kh = kv[:, h * d_keys:(h + 1) * d_keys]
        vh = kv[:, hdk + h * d_keys: hdk + (h + 1) * d_keys]
        s = lax.dot_general(qh, kh, (((1,), (1,)), ((), ())),
                            preferred_element_type=_F32)
        p = _softmax_rows(s)
        o = jnp.dot(p.astype(_BF), vh, preferred_element_type=_F32)
        rep = rep + jnp.dot(o.astype(_BF), ro_w[sl, :],
                            preferred_element_type=_F32)
    rep = rep + ro_b_ref[...]

    rows = G * S
    pwp_tiled = jnp.concatenate([pwp_ref[...]] * G, axis=0)
    x = pwp_tiled + jnp.dot(te_ref[...], rep.astype(_BF),
                            preferred_element_type=_F32)

    hd = d_llm // llm_heads
    scale2 = 1.0 / math.sqrt(hd)
    r = lax.broadcasted_iota(jnp.int32, (rows, rows), 0)
    c = lax.broadcasted_iota(jnp.int32, (rows, rows), 1)
    mask = (c <= r) & ((r // S) == (c // S))

    for l in range(n_layers):
        a = _ln(x, ln1_g_ref[l], ln1_b_ref[l]).astype(_BF)
        qkv = jnp.dot(a, attn_w_ref[l],
                      preferred_element_type=_F32) + attn_b_ref[l]
        pw = proj_w_ref[l]
        att = jnp.zeros((rows, d_llm), _F32)
        for h in range(llm_heads):
            qq = (qkv[:, h * hd:(h + 1) * hd] * scale2).astype(_BF)
            kk = qkv[:, d_llm + h * hd: d_llm + (h + 1) * hd].astype(_BF)
            vv = qkv[:, 2 * d_llm + h * hd: 2 * d_llm + (h + 1) * hd].astype(_BF)
            s = lax.dot_general(qq, kk, (((1,), (1,)), ((), ())),
                                preferred_element_type=_F32)
            s = jnp.where(mask, s, -1e30)
            p = _softmax_rows(s)
            o = jnp.dot(p.astype(_BF), vv, preferred_element_type=_F32)
            att = att + jnp.dot(o.astype(_BF), pw[h * hd:(h + 1) * hd, :],
                                preferred_element_type=_F32)
        x = x + att + proj_b_ref[l]

        a = _ln(x, ln2_g_ref[l], ln2_b_ref[l]).astype(_BF)
        m = jax.nn.gelu((jnp.dot(a, fc_w_ref[l], preferred_element_type=_F32)
                         + fc_b_ref[l]).astype(_BF), approximate=True)
        x = x + jnp.dot(m, mlp_w_ref[l],
                        preferred_element_type=_F32) + mlp_b_ref[l]

    x = _ln(x, lnf_g_ref[...], lnf_b_ref[...])

    x32 = x[:, :d_ffn]
    ys = [jnp.dot(x32, head_w_ref[p_idx], preferred_element_type=_F32)
          for p_idx in range(patch_nums)]
    ycat = jnp.concatenate(ys, axis=0)
    dec = head_b_ref[...] + jnp.dot(sel_ref[...], ycat,
                                    preferred_element_type=_F32)

    recon = dec * std + mean
    out_ref[...] = mk * X + (1.0 - mk) * recon


def _fused_call(xt, mkt, value_emb_w, rq_w, rq_b, kv, ro_w, ro_b, pwp,
                attn_w, attn_b, proj_w, proj_b, ln1_g, ln1_b, ln2_g, ln2_b,
                fc_w, fc_b, mlp_w, mlp_b, lnf_g, lnf_b, head_w, head_b,
                BN, S, T):
    G = 16 if BN % 16 == 0 else (8 if BN % 8 == 0 else 1)
    rows = G * S
    P = _PATCH_NUMS
    rr = jnp.arange(rows)[:, None]
    cc = jnp.arange(P * G)[None, :]
    g_of = rr // S
    t_of = rr % S
    te = ((t_of >= _PROMPT_LEN)
          & (cc == (t_of - _PROMPT_LEN) * G + g_of)).astype(_BF)
    cidx = jnp.arange(P * rows)
    p_of = cidx // rows
    r_of = cidx % rows
    gg = jnp.arange(G)[:, None]
    sel = (r_of[None, :] == gg * S + _PROMPT_LEN + p_of[None, :]).astype(_F32)

    fn = functools.partial(
        _fused_kernel, n_heads=_N_HEADS, d_keys=_D_FFN, n_layers=_N_LAYERS,
        llm_heads=_LLM_HEADS, d_llm=_D_LLM, d_ffn=_D_FFN, patch_nums=P,
        G=G, S=S, T=T)
    w = [value_emb_w, rq_w.astype(_BF), rq_b, kv, ro_w.astype(_BF), ro_b,
         pwp, te,
         attn_w.astype(_BF), attn_b, proj_w.astype(_BF), proj_b,
         ln1_g, ln1_b, ln2_g, ln2_b,
         fc_w.astype(_BF), fc_b, mlp_w.astype(_BF), mlp_b,
         lnf_g, lnf_b, head_w, head_b, sel]
    return pl.pallas_call(
        fn,
        out_shape=jax.ShapeDtypeStruct((BN, T), _F32),
        grid=(BN // G,),
        in_specs=[pl.BlockSpec((G, T), lambda i: (i, 0)),
                  pl.BlockSpec((G, T), lambda i: (i, 0))]
                 + [_full_spec(a) for a in w],
        out_specs=pl.BlockSpec((G, T), lambda i: (i, 0)),
        compiler_params=pltpu.CompilerParams(dimension_semantics=("arbitrary",)),
    )(xt, mkt, *w)


def kernel(value_emb_w, word_emb, map_w, map_b, rq_w, rq_b, rkv_w, rkv_b,
           ro_w, ro_b, wpe, attn_w, attn_b, attn_proj_w, attn_proj_b,
           ln1_g, ln1_b, ln2_g, ln2_b, fc_w, fc_b, mlp_proj_w, mlp_proj_b,
           lnf_g, lnf_b, head_w, head_b, prompt_emb, X, missing_mask):
    B, T, N = X.shape
    BN = B * N
    S = _PROMPT_LEN + _PATCH_NUMS

    xt = jnp.transpose(X, (0, 2, 1)).reshape(BN, T)
    mkt = jnp.transpose(missing_mask, (0, 2, 1)).reshape(BN, T)
    pwp = wpe[:S] + jnp.concatenate(
        [prompt_emb, jnp.zeros((S - _PROMPT_LEN, _D_LLM), _F32)], axis=0)

    kv = _kv_call(map_w, word_emb, map_b, rkv_w, rkv_b)
    imp = _fused_call(xt, mkt, value_emb_w, rq_w, rq_b, kv, ro_w, ro_b, pwp,
                      attn_w, attn_b, attn_proj_w, attn_proj_b,
                      ln1_g, ln1_b, ln2_g, ln2_b, fc_w, fc_b,
                      mlp_proj_w, mlp_proj_b, lnf_g, lnf_b,
                      head_w, head_b, BN, S, T)
    imputed = imp.reshape(B, N, T).transpose(0, 2, 1)
    return {"imputed_data": imputed}


# speedup vs baseline: 1.1686x; 1.1686x over previous
import functools
import math

import jax
import jax.numpy as jnp
from jax import lax
from jax.experimental import pallas as pl
from jax.experimental.pallas import tpu as pltpu

_N_STEPS = 16
_PATCH_SIZE = 4
_PATCH_STRIDE = 4
_PATCH_NUMS = int((_N_STEPS - _PATCH_SIZE) / _PATCH_STRIDE + 2)
_PROMPT_LEN = 8
_D_LLM = 512
_D_MODEL = 32
_D_FFN = 32
_N_HEADS = 4
_LLM_HEADS = 4
_N_LAYERS = 2

_BF = jnp.bfloat16
_F32 = jnp.float32


def _ln(x, g, b, eps=1e-5):
    mu = jnp.mean(x, axis=-1, keepdims=True)
    var = jnp.mean((x - mu) ** 2, axis=-1, keepdims=True)
    return (x - mu) * lax.rsqrt(var + eps) * g + b


def _softmax_rows(s):
    m = jnp.max(s, axis=-1, keepdims=True)
    p = jnp.exp(s - m)
    return p * pl.reciprocal(jnp.sum(p, axis=-1, keepdims=True), approx=True)


def _full_spec(arr):
    return pl.BlockSpec(arr.shape, lambda i, _n=arr.ndim: (0,) * _n)


def _kv_kernel(map_w_ref, word_emb_ref, map_b_ref, rkv_w_ref, rkv_b_ref,
               kv_ref):
    src = jnp.dot(map_w_ref[...], word_emb_ref[...],
                  preferred_element_type=_F32) + map_b_ref[...]
    kv = jnp.dot(src.astype(_BF), rkv_w_ref[...],
                 preferred_element_type=_F32) + rkv_b_ref[...]
    kv_ref[...] = kv.astype(_BF)


def _kv_call(map_w, word_emb, map_b, rkv_w, rkv_b):
    num_tok = map_w.shape[0]
    dqk2 = rkv_w.shape[1]
    ins = [map_w.astype(_BF), word_emb.astype(_BF), map_b,
           rkv_w.astype(_BF), rkv_b]
    return pl.pallas_call(
        _kv_kernel,
        out_shape=jax.ShapeDtypeStruct((num_tok, dqk2), _BF),
        grid=(1,),
        in_specs=[_full_spec(a) for a in ins],
        out_specs=pl.BlockSpec((num_tok, dqk2), lambda i: (0, 0)),
        compiler_params=pltpu.CompilerParams(dimension_semantics=("arbitrary",)),
    )(*ins)


def _reprog_kernel(patch_ref, vemb_ref, rq_w_ref, rq_b_ref, kv_ref,
                   ro_w_ref, ro_b_ref, out_ref, *, n_heads, d_keys):
    patch = patch_ref[...]
    vemb = vemb_ref[...]
    enc = patch[:, 0:1] * vemb[0:1, :]
    for j in range(1, patch.shape[1]):
        enc = enc + patch[:, j:j + 1] * vemb[j:j + 1, :]

    q = jnp.dot(enc.astype(_BF), rq_w_ref[...],
                preferred_element_type=_F32) + rq_b_ref[...]
    kv = kv_ref[...]
    hdk = n_heads * d_keys
    scale = 1.0 / math.sqrt(d_keys)
    ro_w = ro_w_ref[...]
    acc = jnp.zeros(out_ref.shape, _F32)
    for h in range(n_heads):
        sl = slice(h * d_keys, (h + 1) * d_keys)
        qh = q[:, sl].astype(_BF)
        kh = kv[:, h * d_keys:(h + 1) * d_keys]
        vh = kv[:, hdk + h * d_keys: hdk + (h + 1) * d_keys]
        s = lax.dot_general(qh, kh, (((1,), (1,)), ((), ())),
                            preferred_element_type=_F32) * scale
        p = _softmax_rows(s)
        rep = jnp.dot(p.astype(_BF), vh, preferred_element_type=_F32)
        acc = acc + jnp.dot(rep.astype(_BF), ro_w[sl, :],
                            preferred_element_type=_F32)
    out_ref[...] = (acc + ro_b_ref[...]).astype(out_ref.dtype)


def _reprog_call(patches, value_emb_w, rq_w, rq_b, kv, ro_w, ro_b):
    R = patches.shape[0]
    d_llm = ro_w.shape[1]
    tile = 384 if R % 384 == 0 else R
    fn = functools.partial(_reprog_kernel, n_heads=_N_HEADS, d_keys=_D_FFN)
    return pl.pallas_call(
        fn,
        out_shape=jax.ShapeDtypeStruct((R, d_llm), _BF),
        grid=(R // tile,),
        in_specs=[pl.BlockSpec((tile, patches.shape[1]), lambda i: (i, 0)),
                  _full_spec(value_emb_w), _full_spec(rq_w), _full_spec(rq_b),
                  _full_spec(kv), _full_spec(ro_w), _full_spec(ro_b)],
        out_specs=pl.BlockSpec((tile, d_llm), lambda i: (i, 0)),
        compiler_params=pltpu.CompilerParams(dimension_semantics=("parallel",)),
    )(patches, value_emb_w, rq_w.astype(_BF), rq_b, kv,
      ro_w.astype(_BF), ro_b)


def _llm_kernel(enc_ref, xt_ref, mk_ref, pwp_ref, te_ref,
                attn_w_ref, attn_b_ref, proj_w_ref, proj_b_ref,
                ln1_g_ref, ln1_b_ref, ln2_g_ref, ln2_b_ref,
                fc_w_ref, fc_b_ref, mlp_w_ref, mlp_b_ref,
                lnf_g_ref, lnf_b_ref, head_w_ref, head_b_ref, sel_ref,
                out_ref, *, n_layers, n_heads, d_llm, d_ffn, patch_nums,
                G, S, T):
    hd = d_llm // n_heads
    scale = 1.0 / math.sqrt(hd)
    rows = G * S

    pwp_tiled = jnp.concatenate([pwp_ref[...]] * G, axis=0)
    x = pwp_tiled + jnp.dot(te_ref[...], enc_ref[...],
                            preferred_element_type=_F32)
    r = lax.broadcasted_iota(jnp.int32, (rows, rows), 0)
    c = lax.broadcasted_iota(jnp.int32, (rows, rows), 1)
    mask = (c <= r) & ((r // S) == (c // S))

    for l in range(n_layers):
        a = _ln(x, ln1_g_ref[l], ln1_b_ref[l]).astype(_BF)
        qkv = jnp.dot(a, attn_w_ref[l],
                      preferred_element_type=_F32) + attn_b_ref[l]
        pw = proj_w_ref[l]
        att = jnp.zeros((rows, d_llm), _F32)
        for h in range(n_heads):
            q = (qkv[:, h * hd:(h + 1) * hd] * scale).astype(_BF)
            k = qkv[:, d_llm + h * hd: d_llm + (h + 1) * hd].astype(_BF)
            v = qkv[:, 2 * d_llm + h * hd: 2 * d_llm + (h + 1) * hd].astype(_BF)
            s = lax.dot_general(q, k, (((1,), (1,)), ((), ())),
                                preferred_element_type=_F32)
            s = jnp.where(mask, s, -1e30)
            p = _softmax_rows(s)
            o = jnp.dot(p.astype(_BF), v, preferred_element_type=_F32)
            att = att + jnp.dot(o.astype(_BF), pw[h * hd:(h + 1) * hd, :],
                                preferred_element_type=_F32)
        x = x + att + proj_b_ref[l]

        a = _ln(x, ln2_g_ref[l], ln2_b_ref[l]).astype(_BF)
        m = jax.nn.gelu((jnp.dot(a, fc_w_ref[l], preferred_element_type=_F32)
                         + fc_b_ref[l]).astype(_BF), approximate=True)
        x = x + jnp.dot(m, mlp_w_ref[l],
                        preferred_element_type=_F32) + mlp_b_ref[l]

    x = _ln(x, lnf_g_ref[...], lnf_b_ref[...])

    x32 = x[:, :d_ffn]
    ys = [jnp.dot(x32, head_w_ref[p_idx], preferred_element_type=_F32)
          for p_idx in range(patch_nums)]
    ycat = jnp.concatenate(ys, axis=0)
    dec = head_b_ref[...] + jnp.dot(
        sel_ref[...], ycat, preferred_element_type=_F32)

    X = xt_ref[...]
    mk = mk_ref[...]
    cnt = jnp.maximum(jnp.sum(mk, axis=-1, keepdims=True), 1.0)
    mean = jnp.sum(X * mk, axis=-1, keepdims=True) / cnt
    xc = (X - mean) * mk
    std = jnp.sqrt(jnp.sum(xc * xc, axis=-1, keepdims=True) / cnt + 1e-5)
    out_ref[...] = mk * X + (1.0 - mk) * (dec * std + mean)


def _llm_call(enc_llm, xt, mkt, pwp, attn_w, attn_b, proj_w, proj_b,
              ln1_g, ln1_b, ln2_g, ln2_b,
              fc_w, fc_b, mlp_w, mlp_b, lnf_g, lnf_b, head_w, head_b,
              BN, S, T):
    G = 16 if BN % 16 == 0 else (8 if BN % 8 == 0 else 1)
    d_llm = enc_llm.shape[1]
    rows = G * S
    P = _PATCH_NUMS
    rr = jnp.arange(rows)[:, None]
    cc = jnp.arange(G * P)[None, :]
    g_of = rr // S
    t_of = rr % S
    te = ((t_of >= _PROMPT_LEN)
          & (cc == g_of * P + (t_of - _PROMPT_LEN))).astype(_BF)
    cidx = jnp.arange(P * rows)
    p_of = cidx // rows
    r_of = cidx % rows
    gg = jnp.arange(G)[:, None]
    sel = (r_of[None, :] == gg * S + _PROMPT_LEN + p_of[None, :]
           ).astype(_F32)
    fn = functools.partial(
        _llm_kernel, n_layers=_N_LAYERS, n_heads=_LLM_HEADS, d_llm=d_llm,
        d_ffn=_D_FFN, patch_nums=P, G=G, S=S, T=T)
    w = [attn_w.astype(_BF), attn_b, proj_w.astype(_BF), proj_b,
         ln1_g, ln1_b, ln2_g, ln2_b,
         fc_w.astype(_BF), fc_b, mlp_w.astype(_BF), mlp_b,
         lnf_g, lnf_b, head_w, head_b, sel]
    return pl.pallas_call(
        fn,
        out_shape=jax.ShapeDtypeStruct((BN, T), _F32),
        grid=(BN // G,),
        in_specs=[pl.BlockSpec((G * P, d_llm), lambda i: (i, 0)),
                  pl.BlockSpec((G, T), lambda i: (i, 0)),
                  pl.BlockSpec((G, T), lambda i: (i, 0)),
                  _full_spec(pwp), _full_spec(te)]
                 + [_full_spec(a) for a in w],
        out_specs=pl.BlockSpec((G, T), lambda i: (i, 0)),
        compiler_params=pltpu.CompilerParams(dimension_semantics=("parallel",)),
    )(enc_llm, xt, mkt, pwp, te, *w)


def kernel(value_emb_w, word_emb, map_w, map_b, rq_w, rq_b, rkv_w, rkv_b,
           ro_w, ro_b, wpe, attn_w, attn_b, attn_proj_w, attn_proj_b,
           ln1_g, ln1_b, ln2_g, ln2_b, fc_w, fc_b, mlp_proj_w, mlp_proj_b,
           lnf_g, lnf_b, head_w, head_b, prompt_emb, X, missing_mask):
    B, T, N = X.shape
    BN = B * N
    P = _PATCH_NUMS
    S = _PROMPT_LEN + P

    xt = jnp.transpose(X, (0, 2, 1)).reshape(BN, T)
    mkt = jnp.transpose(missing_mask, (0, 2, 1)).reshape(BN, T)
    cnt = jnp.maximum(jnp.sum(mkt, axis=1, keepdims=True), 1.0)
    means = jnp.sum(xt * mkt, axis=1, keepdims=True) / cnt
    x = (xt - means) * mkt
    stdev = jnp.sqrt(jnp.sum(x * x, axis=1, keepdims=True) / cnt + 1e-5)
    x = x / stdev

    x = jnp.concatenate(
        [x, jnp.repeat(x[:, -1:], _PATCH_STRIDE, axis=1)], axis=1)
    idx = (jnp.arange(P)[:, None] * _PATCH_STRIDE
           + jnp.arange(_PATCH_SIZE)[None, :])
    patches = x[:, idx].reshape(BN * P, _PATCH_SIZE)

    kv = _kv_call(map_w, word_emb, map_b, rkv_w, rkv_b)
    enc_llm = _reprog_call(patches, value_emb_w, rq_w, rq_b, kv, ro_w, ro_b)

    pwp = wpe[:S] + jnp.concatenate(
        [prompt_emb, jnp.zeros((S - _PROMPT_LEN, _D_LLM), _F32)], axis=0)

    imp = _llm_call(enc_llm, xt, mkt, pwp, attn_w, attn_b,
                    attn_proj_w, attn_proj_b,
                    ln1_g, ln1_b, ln2_g, ln2_b, fc_w, fc_b,
                    mlp_proj_w, mlp_proj_b, lnf_g, lnf_b,
                    head_w, head_b, BN, S, T)
    imputed = imp.reshape(B, N, T).transpose(0, 2, 1)
    return {"imputed_data": imputed}
